# Initial kernel scaffold; baseline (speedup 1.0000x reference)
import functools

import jax
import jax.numpy as jnp
from jax import lax
from jax.experimental import pallas as pl
from jax.experimental.pallas import tpu as pltpu
from jax.experimental.pallas import tpu_sc as plsc

C = 8
NC, NS = 2, 16
NW = NC * NS
S_CHUNK = 8192


def _tr_in_body(eye_ref, x_ref, o_ref):
    o_ref[...] = lax.dot_general(
        x_ref[...], eye_ref[...], (((0,), (0,)), ((), ())),
        preferred_element_type=jnp.float32)


def _tr_out_body(eye_ref, x_ref, o_ref):
    o_ref[0] = lax.dot_general(
        eye_ref[...], x_ref[0], (((1,), (1,)), ((), ())),
        preferred_element_type=jnp.float32)


def _sc_gather(table, idx, n):
    per_w = n // NW
    n_chunks = per_w // S_CHUNK
    mesh = plsc.VectorSubcoreMesh(core_axis_name="c", subcore_axis_name="s")

    @functools.partial(
        pl.kernel,
        mesh=mesh,
        out_type=jax.ShapeDtypeStruct((n, C), jnp.float32),
        scratch_types=[
            pltpu.VMEM((S_CHUNK,), jnp.int32),
            pltpu.VMEM((S_CHUNK, C), jnp.float32),
            pltpu.SemaphoreType.DMA,
        ],
    )
    def k(table_hbm, idx_hbm, out_hbm, idx_v, rows_v, sem):
        wid = lax.axis_index("s") * NC + lax.axis_index("c")
        base = wid * per_w

        def body(j, carry):
            off = base + j * S_CHUNK
            pltpu.sync_copy(idx_hbm.at[pl.ds(off, S_CHUNK)], idx_v)
            pltpu.async_copy(table_hbm.at[idx_v], rows_v, sem).wait()
            pltpu.sync_copy(rows_v, out_hbm.at[pl.ds(off, S_CHUNK)])
            return carry

        lax.fori_loop(0, n_chunks, body, 0)

    return k(table, idx)


def kernel(inputs, texture):
    ids = inputs
    b, h, w = ids.shape
    size = texture.shape[2]
    n = b * h * w
    hw = h * w
    eye = jnp.eye(C, dtype=jnp.float32)

    cw = 8000
    tex2d = texture.reshape(C, size)
    table = pl.pallas_call(
        _tr_in_body,
        grid=(size // cw,),
        in_specs=[
            pl.BlockSpec((C, C), lambda i: (0, 0)),
            pl.BlockSpec((C, cw), lambda i: (0, i)),
        ],
        out_specs=pl.BlockSpec((cw, C), lambda i: (i, 0)),
        out_shape=jax.ShapeDtypeStruct((size, C), jnp.float32),
    )(eye, tex2d)

    rows = _sc_gather(table, ids.reshape(n), n)

    ck = 8192
    out = pl.pallas_call(
        _tr_out_body,
        grid=(b, hw // ck),
        in_specs=[
            pl.BlockSpec((C, C), lambda i, j: (0, 0)),
            pl.BlockSpec((1, ck, C), lambda i, j: (i, j, 0)),
        ],
        out_specs=pl.BlockSpec((1, C, ck), lambda i, j: (i, 0, j)),
        out_shape=jax.ShapeDtypeStruct((b, C, hw), jnp.float32),
    )(eye, rows.reshape(b, hw, C))

    return out.reshape(b, C, h, w)


# speedup vs baseline: 2.2448x; 2.2448x over previous
import functools

import jax
import jax.numpy as jnp
from jax import lax
from jax.experimental import pallas as pl
from jax.experimental.pallas import tpu as pltpu
from jax.experimental.pallas import tpu_sc as plsc

C = 8
NC, NS = 2, 16
NW = NC * NS
S_CHUNK = 8192


def _tr_in_body(eye_ref, x_ref, o_ref):
    for k in range(x_ref.shape[1]):
        o_ref[k] = lax.dot_general(
            x_ref[:, k, :], eye_ref[...], (((0,), (0,)), ((), ())),
            precision=lax.Precision.HIGHEST,
            preferred_element_type=jnp.float32)


def _tr_out_body(eye_ref, x_ref, o_ref):
    o_ref[0] = lax.dot_general(
        eye_ref[...], x_ref[0], (((1,), (1,)), ((), ())),
        precision=lax.Precision.HIGHEST,
        preferred_element_type=jnp.float32)


def _sc_gather(table, idx, n):
    per_w = n // NW
    n_chunks = per_w // S_CHUNK
    mesh = plsc.VectorSubcoreMesh(core_axis_name="c", subcore_axis_name="s")

    @functools.partial(
        pl.kernel,
        mesh=mesh,
        out_type=jax.ShapeDtypeStruct((n, C), jnp.float32),
        scratch_types=[
            pltpu.VMEM((S_CHUNK,), jnp.int32),
            pltpu.VMEM((S_CHUNK, C), jnp.float32),
            pltpu.SemaphoreType.DMA,
        ],
        compiler_params=pltpu.CompilerParams(use_tc_tiling_on_sc=False),
    )
    def k(table_hbm, idx_hbm, out_hbm, idx_v, rows_v, sem):
        wid = lax.axis_index("s") * NC + lax.axis_index("c")
        base = wid * per_w

        def body(j, carry):
            off = base + j * S_CHUNK
            pltpu.sync_copy(idx_hbm.at[pl.ds(off, S_CHUNK)], idx_v)
            pltpu.async_copy(table_hbm.at[idx_v], rows_v, sem).wait()
            pltpu.sync_copy(rows_v, out_hbm.at[pl.ds(off, S_CHUNK)])
            return carry

        lax.fori_loop(0, n_chunks, body, 0)

    return k(table, idx)


def kernel(inputs, texture):
    ids = inputs
    b, h, w = ids.shape
    size = texture.shape[2]
    n = b * h * w
    hw = h * w
    eye = jnp.eye(C, dtype=jnp.float32)

    ph, pl_minor, bh = 1000, size // 1000, 8
    tex3d = texture.reshape(C, ph, pl_minor)
    table3 = pl.pallas_call(
        _tr_in_body,
        grid=(ph // bh,),
        in_specs=[
            pl.BlockSpec((C, C), lambda i: (0, 0)),
            pl.BlockSpec((C, bh, pl_minor), lambda i: (0, i, 0)),
        ],
        out_specs=pl.BlockSpec((bh, pl_minor, C), lambda i: (i, 0, 0)),
        out_shape=jax.ShapeDtypeStruct((ph, pl_minor, C), jnp.float32),
    )(eye, tex3d)
    table = table3.reshape(size, C)

    rows = _sc_gather(table, ids.reshape(n), n)

    ck = 8192
    out = pl.pallas_call(
        _tr_out_body,
        grid=(b, hw // ck),
        in_specs=[
            pl.BlockSpec((C, C), lambda i, j: (0, 0)),
            pl.BlockSpec((1, ck, C), lambda i, j: (i, j, 0)),
        ],
        out_specs=pl.BlockSpec((1, C, ck), lambda i, j: (i, 0, j)),
        out_shape=jax.ShapeDtypeStruct((b, C, hw), jnp.float32),
    )(eye, rows.reshape(b, hw, C))

    return out.reshape(b, C, h, w)


# speedup vs baseline: 5.6090x; 2.4986x over previous
import functools

import jax
import jax.numpy as jnp
from jax import lax
from jax.experimental import pallas as pl
from jax.experimental.pallas import tpu as pltpu
from jax.experimental.pallas import tpu_sc as plsc

C = 8
NC, NS = 2, 16
NW = NC * NS
L = 16
P_BUILD = 4000
S_CHUNK = 2048


def _sc_all(tex1d, ids1d, n, size, hw):
    per_w = n // NW
    n_g = per_w // S_CHUNK
    nb = size // P_BUILD
    full, extra = nb // NS, nb % NS
    chw = C * hw
    mesh = plsc.VectorSubcoreMesh(core_axis_name="c", subcore_axis_name="s")

    @functools.partial(
        pl.kernel,
        mesh=mesh,
        out_type=jax.ShapeDtypeStruct((n * C,), jnp.float32),
        scratch_types=[
            pltpu.HBM((NC, size, C), jnp.float32),
            pltpu.VMEM((C, P_BUILD), jnp.float32),
            pltpu.VMEM((P_BUILD, C), jnp.float32),
            pltpu.VMEM((S_CHUNK,), jnp.int32),
            pltpu.VMEM((S_CHUNK, C), jnp.float32),
            pltpu.VMEM((C, S_CHUNK), jnp.float32),
            pltpu.SemaphoreType.DMA,
        ],
        compiler_params=pltpu.CompilerParams(use_tc_tiling_on_sc=False,
                                             needs_layout_passes=False),
    )
    def k(tex_hbm, ids_hbm, out_hbm, table_hbm, tin_v, tbuf_v, idx_v,
          rows_v, cm_v, sem):
        cid = lax.axis_index("c")
        sid = lax.axis_index("s")
        iota = lax.iota(jnp.int32, L)
        ch_splat = [jnp.full((L,), ch, jnp.int32) for ch in range(C)]

        def build(kk, carry):
            off = (sid + NS * kk) * P_BUILD
            for ch in range(C):
                pltpu.sync_copy(tex_hbm.at[pl.ds(ch * size + off, P_BUILD)],
                                tin_v.at[ch])

            def tr(g, c2):
                p_idx = g * L + iota
                for ch in range(C):
                    vals = tin_v[ch, pl.ds(g * L, L)]
                    plsc.store_scatter(tbuf_v, [p_idx, ch_splat[ch]], vals)
                return c2

            lax.fori_loop(0, P_BUILD // L, tr, 0)
            pltpu.sync_copy(tbuf_v, table_hbm.at[cid, pl.ds(off, P_BUILD)])
            return carry

        nk = jnp.where(sid < extra, full + 1, full)
        lax.fori_loop(0, nk, build, 0)
        plsc.subcore_barrier()

        wid = sid * NC + cid
        base = wid * per_w
        bb = wid // (hw // per_w)
        col0 = (wid % (hw // per_w)) * per_w

        def gather(j, carry):
            off = base + j * S_CHUNK
            pltpu.sync_copy(ids_hbm.at[pl.ds(off, S_CHUNK)], idx_v)
            pltpu.async_copy(table_hbm.at[cid].at[idx_v], rows_v, sem).wait()

            def tr(g, c2):
                s_idx = g * L + iota
                for ch in range(C):
                    vals = plsc.load_gather(rows_v, [s_idx, ch_splat[ch]])
                    cm_v[ch, pl.ds(g * L, L)] = vals
                return c2

            lax.fori_loop(0, S_CHUNK // L, tr, 0)
            obase = bb * chw + col0 + j * S_CHUNK
            for ch in range(C):
                pltpu.sync_copy(cm_v.at[ch],
                                out_hbm.at[pl.ds(obase + ch * hw, S_CHUNK)])
            return carry

        lax.fori_loop(0, n_g, gather, 0)

    return k(tex1d, ids1d)


def kernel(inputs, texture):
    ids = inputs
    b, h, w = ids.shape
    size = texture.shape[2]
    n = b * h * w
    hw = h * w
    out1d = _sc_all(texture.reshape(C * size), ids.reshape(n), n, size, hw)
    return out1d.reshape(b, C, h, w)


# speedup vs baseline: 5.9681x; 1.0640x over previous
import functools

import jax
import jax.numpy as jnp
from jax import lax
from jax.experimental import pallas as pl
from jax.experimental.pallas import tpu as pltpu
from jax.experimental.pallas import tpu_sc as plsc

C = 8
NC, NS = 2, 16
NW = NC * NS
L = 16
P_BUILD = 2000
S_CHUNK = 4096


def _sc_all(tex1d, ids1d, n, size, b, h, w):
    hw = h * w
    per_w = n // NW
    n_g = per_w // S_CHUNK
    nb = size // P_BUILD
    full, extra = nb // NS, nb % NS
    rows_per_chunk = S_CHUNK // w
    mesh = plsc.VectorSubcoreMesh(core_axis_name="c", subcore_axis_name="s")

    @functools.partial(
        pl.kernel,
        mesh=mesh,
        out_type=jax.ShapeDtypeStruct((b, C, h, w), jnp.float32),
        scratch_types=[
            pltpu.HBM((NC, size, C), jnp.float32),
            pltpu.VMEM((C, P_BUILD), jnp.float32),
            pltpu.VMEM((P_BUILD, C), jnp.float32),
            pltpu.VMEM((S_CHUNK,), jnp.int32),
            pltpu.VMEM((S_CHUNK, C), jnp.float32),
            pltpu.VMEM((C, rows_per_chunk, w), jnp.float32),
            pltpu.SemaphoreType.DMA,
        ],
        compiler_params=pltpu.CompilerParams(use_tc_tiling_on_sc=False,
                                             needs_layout_passes=False),
    )
    def k(tex_hbm, ids_hbm, out_hbm, table_hbm, tin_v, tbuf_v, idx_v,
          rows_v, cm_v, sem):
        cid = lax.axis_index("c")
        sid = lax.axis_index("s")
        iota = lax.iota(jnp.int32, L)
        ch_splat = [jnp.full((L,), ch, jnp.int32) for ch in range(C)]

        def build(kk, carry):
            off = (sid + NS * kk) * P_BUILD
            hs = [pltpu.async_copy(
                      tex_hbm.at[pl.ds(ch * size + off, P_BUILD)],
                      tin_v.at[ch], sem) for ch in range(C)]
            for hh in hs:
                hh.wait()

            def tr(g, c2):
                p_idx = g * L + iota
                for ch in range(C):
                    vals = tin_v[ch, pl.ds(g * L, L)]
                    plsc.store_scatter(tbuf_v, [p_idx, ch_splat[ch]], vals)
                return c2

            lax.fori_loop(0, P_BUILD // L, tr, 0)
            pltpu.sync_copy(tbuf_v, table_hbm.at[cid, pl.ds(off, P_BUILD)])
            return carry

        nk = jnp.where(sid < extra, full + 1, full)
        lax.fori_loop(0, nk, build, 0)
        plsc.subcore_barrier()

        wid = sid * NC + cid
        base = wid * per_w
        bb = wid // (hw // per_w)
        col0 = (wid % (hw // per_w)) * per_w
        h0w = col0 // w
        wtiles = w // 128

        def gather(j, carry):
            off = base + j * S_CHUNK
            pltpu.sync_copy(ids_hbm.at[pl.ds(off, S_CHUNK)], idx_v)
            pltpu.async_copy(table_hbm.at[cid].at[idx_v], rows_v, sem).wait()

            gpl = 128 // L
            for i in range(rows_per_chunk):
                t_blk = (i * w) // 1024

                def tr(gg, c2, i=i, t_blk=t_blk):
                    s_idx = i * w + gg * L + iota
                    for ch in range(C):
                        vals = plsc.load_gather(rows_v,
                                                [s_idx, ch_splat[ch]])
                        cm_v[ch, i, pl.ds(gg * L, L)] = vals
                    return c2

                lax.fori_loop(0, w // L, tr, 0)

            hrow = h0w + j * rows_per_chunk
            hs = [pltpu.async_copy(
                      cm_v.at[ch],
                      out_hbm.at[bb, ch, pl.ds(hrow, rows_per_chunk), :],
                      sem) for ch in range(C)]
            for hh in hs:
                hh.wait()
            return carry

        lax.fori_loop(0, n_g, gather, 0)

    return k(tex1d, ids1d)


def kernel(inputs, texture):
    ids = inputs
    b, h, w = ids.shape
    size = texture.shape[2]
    n = b * h * w
    return _sc_all(texture.reshape(C * size), ids.reshape(n), n, size,
                   b, h, w)


# speedup vs baseline: 5.9712x; 1.0005x over previous
import functools

import jax
import jax.numpy as jnp
from jax import lax
from jax.experimental import pallas as pl
from jax.experimental.pallas import tpu as pltpu
from jax.experimental.pallas import tpu_sc as plsc

C = 8
NC, NS = 2, 16
NW = NC * NS
L = 16
P_BUILD = 2000
S_CHUNK = 4096


def _sc_all(tex3d, ids1d, n, size, b, h, w):
    hw = h * w
    per_w = n // NW
    n_g = per_w // S_CHUNK
    nb = size // P_BUILD
    full, extra = nb // NS, nb % NS
    rows_per_chunk = S_CHUNK // w
    mesh = plsc.VectorSubcoreMesh(core_axis_name="c", subcore_axis_name="s")

    @functools.partial(
        pl.kernel,
        mesh=mesh,
        out_type=jax.ShapeDtypeStruct((b, C, h, w), jnp.float32),
        scratch_types=[
            pltpu.HBM((NC, size, C), jnp.float32),
            pltpu.VMEM((C, P_BUILD), jnp.float32),
            pltpu.VMEM((P_BUILD, C), jnp.float32),
            pltpu.VMEM((S_CHUNK,), jnp.int32),
            pltpu.VMEM((S_CHUNK, C), jnp.float32),
            pltpu.VMEM((C, rows_per_chunk, w), jnp.float32),
            pltpu.SemaphoreType.DMA,
        ],
        compiler_params=pltpu.CompilerParams(use_tc_tiling_on_sc=False,
                                             needs_layout_passes=False),
    )
    def k(tex_hbm, ids_hbm, out_hbm, table_hbm, tin_v, tbuf_v, idx_v,
          rows_v, cm_v, sem):
        cid = lax.axis_index("c")
        sid = lax.axis_index("s")
        iota = lax.iota(jnp.int32, L)
        ch_splat = [jnp.full((L,), ch, jnp.int32) for ch in range(C)]

        def build(kk, carry):
            off = (sid + NS * kk) * P_BUILD
            hs = [pltpu.async_copy(
                      tex_hbm.at[0, ch, pl.ds(off, P_BUILD)],
                      tin_v.at[ch], sem) for ch in range(C)]
            for hh in hs:
                hh.wait()

            def tr(g, c2):
                p_idx = g * L + iota
                for ch in range(C):
                    vals = tin_v[ch, pl.ds(g * L, L)]
                    plsc.store_scatter(tbuf_v, [p_idx, ch_splat[ch]], vals)
                return c2

            lax.fori_loop(0, P_BUILD // L, tr, 0)
            pltpu.sync_copy(tbuf_v, table_hbm.at[cid, pl.ds(off, P_BUILD)])
            return carry

        nk = jnp.where(sid < extra, full + 1, full)
        lax.fori_loop(0, nk, build, 0)
        plsc.subcore_barrier()

        wid = sid * NC + cid
        base = wid * per_w
        bb = wid // (hw // per_w)
        col0 = (wid % (hw // per_w)) * per_w
        h0w = col0 // w
        wtiles = w // 128

        def gather(j, carry):
            off = base + j * S_CHUNK
            pltpu.sync_copy(ids_hbm.at[pl.ds(off, S_CHUNK)], idx_v)
            pltpu.async_copy(table_hbm.at[cid].at[idx_v], rows_v, sem).wait()

            gpl = 128 // L
            for i in range(rows_per_chunk):
                t_blk = (i * w) // 1024

                def tr(gg, c2, i=i, t_blk=t_blk):
                    s_idx = i * w + gg * L + iota
                    for ch in range(C):
                        vals = plsc.load_gather(rows_v,
                                                [s_idx, ch_splat[ch]])
                        cm_v[ch, i, pl.ds(gg * L, L)] = vals
                    return c2

                lax.fori_loop(0, w // L, tr, 0)

            hrow = h0w + j * rows_per_chunk
            hs = [pltpu.async_copy(
                      cm_v.at[ch],
                      out_hbm.at[bb, ch, pl.ds(hrow, rows_per_chunk), :],
                      sem) for ch in range(C)]
            for hh in hs:
                hh.wait()
            return carry

        lax.fori_loop(0, n_g, gather, 0)

    return k(tex3d, ids1d)


def kernel(inputs, texture):
    ids = inputs
    b, h, w = ids.shape
    size = texture.shape[2]
    n = b * h * w
    return _sc_all(texture, ids.reshape(n), n, size, b, h, w)


# speedup vs baseline: 11.3279x; 1.8971x over previous
import functools

import jax
import jax.numpy as jnp
from jax import lax
from jax.experimental import pallas as pl
from jax.experimental.pallas import tpu as pltpu
from jax.experimental.pallas import tpu_sc as plsc

C = 8
NC, NS = 2, 16
NW = NC * NS
L = 16
P_BUILD = 2000
S_CHUNK = 4096


def _sc_all(tex_cs, ids1d, n, size, b, h, w):
    hw = h * w
    per_w = n // NW
    n_g = per_w // S_CHUNK
    nb = size // P_BUILD
    full, extra = nb // NS, nb % NS
    rows_per_chunk = S_CHUNK // w
    mesh = plsc.VectorSubcoreMesh(core_axis_name="c", subcore_axis_name="s")

    @functools.partial(
        pl.kernel,
        mesh=mesh,
        out_type=jax.ShapeDtypeStruct((b, C, h, w), jnp.float32),
        scratch_types=[
            pltpu.HBM((NC, size, C), jnp.float32),
            pltpu.VMEM((C, P_BUILD), jnp.float32),
            pltpu.VMEM((P_BUILD, C), jnp.float32),
            pltpu.VMEM((S_CHUNK,), jnp.int32),
            pltpu.VMEM((S_CHUNK, C), jnp.float32),
            pltpu.VMEM((C, rows_per_chunk, w), jnp.float32),
            pltpu.SemaphoreType.DMA,
        ],
        compiler_params=pltpu.CompilerParams(use_tc_tiling_on_sc=False,
                                             needs_layout_passes=False),
    )
    def k(t0, t1, t2, t3, t4, t5, t6, t7, ids_hbm, out_hbm, table_hbm,
          tin_v, tbuf_v, idx_v, rows_v, cm_v, sem):
        tex_refs = [t0, t1, t2, t3, t4, t5, t6, t7]
        cid = lax.axis_index("c")
        sid = lax.axis_index("s")
        iota = lax.iota(jnp.int32, L)
        ch_splat = [jnp.full((L,), ch, jnp.int32) for ch in range(C)]

        def build(kk, carry):
            off = (sid + NS * kk) * P_BUILD
            hs = [pltpu.async_copy(
                      tex_refs[ch].at[pl.ds(off, P_BUILD)],
                      tin_v.at[ch], sem) for ch in range(C)]
            for hh in hs:
                hh.wait()

            def tr(g, c2):
                p_idx = g * L + iota
                for ch in range(C):
                    vals = tin_v[ch, pl.ds(g * L, L)]
                    plsc.store_scatter(tbuf_v, [p_idx, ch_splat[ch]], vals)
                return c2

            lax.fori_loop(0, P_BUILD // L, tr, 0)
            pltpu.sync_copy(tbuf_v, table_hbm.at[cid, pl.ds(off, P_BUILD)])
            return carry

        nk = jnp.where(sid < extra, full + 1, full)
        lax.fori_loop(0, nk, build, 0)
        plsc.subcore_barrier()

        wid = sid * NC + cid
        base = wid * per_w
        bb = wid // (hw // per_w)
        col0 = (wid % (hw // per_w)) * per_w
        h0w = col0 // w
        wtiles = w // 128

        def gather(j, carry):
            off = base + j * S_CHUNK
            pltpu.sync_copy(ids_hbm.at[pl.ds(off, S_CHUNK)], idx_v)
            pltpu.async_copy(table_hbm.at[cid].at[idx_v], rows_v, sem).wait()

            gpl = 128 // L
            for i in range(rows_per_chunk):
                t_blk = (i * w) // 1024

                def tr(gg, c2, i=i, t_blk=t_blk):
                    s_idx = i * w + gg * L + iota
                    for ch in range(C):
                        vals = plsc.load_gather(rows_v,
                                                [s_idx, ch_splat[ch]])
                        cm_v[ch, i, pl.ds(gg * L, L)] = vals
                    return c2

                lax.fori_loop(0, w // L, tr, 0)

            hrow = h0w + j * rows_per_chunk
            hs = [pltpu.async_copy(
                      cm_v.at[ch],
                      out_hbm.at[bb, ch, pl.ds(hrow, rows_per_chunk), :],
                      sem) for ch in range(C)]
            for hh in hs:
                hh.wait()
            return carry

        lax.fori_loop(0, n_g, gather, 0)

    return k(*tex_cs, ids1d)


def kernel(inputs, texture):
    ids = inputs
    b, h, w = ids.shape
    size = texture.shape[2]
    n = b * h * w
    tex_cs = [texture[0, c] for c in range(C)]
    return _sc_all(tex_cs, ids.reshape(n), n, size, b, h, w)


# speedup vs baseline: 11.7011x; 1.0329x over previous
import functools

import jax
import jax.numpy as jnp
from jax import lax
from jax.experimental import pallas as pl
from jax.experimental.pallas import tpu as pltpu
from jax.experimental.pallas import tpu_sc as plsc

C = 8
NC, NS = 2, 16
NW = NC * NS
L = 16
P_BUILD = 4096
S_CHUNK = 4096


def _sc_all(tex_cs, ids1d, n, size, b, h, w):
    hw = h * w
    per_w = n // NW
    n_g = per_w // S_CHUNK
    nfull = size // P_BUILD
    rem = size - nfull * P_BUILD
    full, extra = nfull // NS, nfull % NS
    rows_per_chunk = S_CHUNK // w
    mesh = plsc.VectorSubcoreMesh(core_axis_name="c", subcore_axis_name="s")

    @functools.partial(
        pl.kernel,
        mesh=mesh,
        out_type=jax.ShapeDtypeStruct((b, C, h, w), jnp.float32),
        scratch_types=[
            pltpu.HBM((NC, size, C), jnp.float32),
            pltpu.VMEM((C, P_BUILD), jnp.float32),
            pltpu.VMEM((S_CHUNK,), jnp.int32),
            pltpu.VMEM((S_CHUNK, C), jnp.float32),
            pltpu.VMEM((C, rows_per_chunk, w), jnp.float32),
            pltpu.SemaphoreType.DMA,
        ],
        compiler_params=pltpu.CompilerParams(use_tc_tiling_on_sc=False,
                                             needs_layout_passes=False),
    )
    def k(t0, t1, t2, t3, t4, t5, t6, t7, ids_hbm, out_hbm, table_hbm,
          tin_v, idx_v, rows_v, cm_v, sem):
        tex_refs = [t0, t1, t2, t3, t4, t5, t6, t7]
        cid = lax.axis_index("c")
        sid = lax.axis_index("s")
        iota = lax.iota(jnp.int32, L)
        ch_splat = [jnp.full((L,), ch, jnp.int32) for ch in range(C)]

        def drain_table_write():
            pltpu.make_async_copy(
                rows_v, table_hbm.at[cid, pl.ds(0, P_BUILD)], sem).wait()

        def build(kk, carry):
            off = (sid + NS * kk) * P_BUILD
            hs = [pltpu.async_copy(
                      tex_refs[ch].at[pl.ds(off, P_BUILD)],
                      tin_v.at[ch], sem) for ch in range(C)]
            for hh in hs:
                hh.wait()

            @pl.when(kk > 0)
            def _():
                drain_table_write()

            def tr(g, c2):
                p_idx = g * L + iota
                for ch in range(C):
                    vals = tin_v[ch, pl.ds(g * L, L)]
                    plsc.store_scatter(rows_v, [p_idx, ch_splat[ch]], vals)
                return c2

            lax.fori_loop(0, P_BUILD // L, tr, 0)
            pltpu.async_copy(
                rows_v, table_hbm.at[cid, pl.ds(off, P_BUILD)], sem)
            return carry

        nk = jnp.where(sid < extra, full + 1, full)
        lax.fori_loop(0, nk, build, 0)
        drain_table_write()

        if rem:
            @pl.when(sid == extra)
            def _():
                roff = nfull * P_BUILD
                hs = [pltpu.async_copy(
                          tex_refs[ch].at[pl.ds(roff, rem)],
                          tin_v.at[ch, pl.ds(0, rem)], sem)
                      for ch in range(C)]
                for hh in hs:
                    hh.wait()

                def trr(g, c2):
                    p_idx = g * L + iota
                    for ch in range(C):
                        vals = tin_v[ch, pl.ds(g * L, L)]
                        plsc.store_scatter(rows_v, [p_idx, ch_splat[ch]],
                                           vals)
                    return c2

                lax.fori_loop(0, rem // L, trr, 0)
                pltpu.sync_copy(rows_v.at[pl.ds(0, rem)],
                                table_hbm.at[cid, pl.ds(roff, rem)])

        plsc.subcore_barrier()

        wid = sid * NC + cid
        base = wid * per_w
        bb = wid // (hw // per_w)
        col0 = (wid % (hw // per_w)) * per_w
        h0w = col0 // w

        def drain_out_writes():
            for ch in range(C):
                pltpu.make_async_copy(
                    cm_v.at[ch],
                    out_hbm.at[bb, ch, pl.ds(h0w, rows_per_chunk), :],
                    sem).wait()

        def gather(j, carry):
            off = base + j * S_CHUNK
            pltpu.sync_copy(ids_hbm.at[pl.ds(off, S_CHUNK)], idx_v)
            pltpu.async_copy(table_hbm.at[cid].at[idx_v], rows_v, sem).wait()

            @pl.when(j > 0)
            def _():
                drain_out_writes()

            for i in range(rows_per_chunk):

                def tr(gg, c2, i=i):
                    s_idx = i * w + gg * L + iota
                    for ch in range(C):
                        vals = plsc.load_gather(rows_v,
                                                [s_idx, ch_splat[ch]])
                        cm_v[ch, i, pl.ds(gg * L, L)] = vals
                    return c2

                lax.fori_loop(0, w // L, tr, 0)

            hrow = h0w + j * rows_per_chunk
            for ch in range(C):
                pltpu.async_copy(
                    cm_v.at[ch],
                    out_hbm.at[bb, ch, pl.ds(hrow, rows_per_chunk), :],
                    sem)
            return carry

        lax.fori_loop(0, n_g, gather, 0)
        drain_out_writes()

    return k(*tex_cs, ids1d)


def kernel(inputs, texture):
    ids = inputs
    b, h, w = ids.shape
    size = texture.shape[2]
    n = b * h * w
    tex_cs = [texture[0, c] for c in range(C)]
    return _sc_all(tex_cs, ids.reshape(n), n, size, b, h, w)


# speedup vs baseline: 16.7097x; 1.4280x over previous
import functools

import jax
import jax.numpy as jnp
from jax import lax
from jax.experimental import pallas as pl
from jax.experimental.pallas import tpu as pltpu
from jax.experimental.pallas import tpu_sc as plsc

C = 8
NC, NS = 2, 16
NW = NC * NS
L = 16
P_BUILD = 4096
S_CHUNK = 4096


def _sc_all(tex_cs, ids1d, n, size, b, h, w):
    hw = h * w
    per_w = n // NW
    n_g = per_w // S_CHUNK
    nfull = size // P_BUILD
    rem = size - nfull * P_BUILD
    full, extra = nfull // NS, nfull % NS
    rows_per_chunk = S_CHUNK // w
    mesh = plsc.VectorSubcoreMesh(core_axis_name="c", subcore_axis_name="s")

    @functools.partial(
        pl.kernel,
        mesh=mesh,
        out_type=jax.ShapeDtypeStruct((b, C, h, w), jnp.float32),
        scratch_types=[
            pltpu.HBM((NC, size, C), jnp.float32),
            pltpu.VMEM((C, P_BUILD), jnp.float32),
            pltpu.VMEM((S_CHUNK,), jnp.int32),
            pltpu.VMEM((S_CHUNK, C), jnp.float32),
            pltpu.VMEM((C, S_CHUNK // 512, 512), jnp.float32),
            pltpu.SemaphoreType.DMA,
        ],
        compiler_params=pltpu.CompilerParams(use_tc_tiling_on_sc=False,
                                             needs_layout_passes=False),
    )
    def k(t0, t1, t2, t3, t4, t5, t6, t7, ids_hbm, out_hbm, table_hbm,
          tin_v, idx_v, rows_v, cm_v2, sem):
        tex_refs = [t0, t1, t2, t3, t4, t5, t6, t7]
        cid = lax.axis_index("c")
        sid = lax.axis_index("s")
        iota = lax.iota(jnp.int32, L)
        ch_splat = [jnp.full((L,), ch, jnp.int32) for ch in range(C)]

        def drain_table_write():
            pltpu.make_async_copy(
                rows_v, table_hbm.at[cid, pl.ds(0, P_BUILD)], sem).wait()

        def build(kk, carry):
            off = (sid + NS * kk) * P_BUILD
            hs = [pltpu.async_copy(
                      tex_refs[ch].at[pl.ds(off, P_BUILD)],
                      tin_v.at[ch], sem) for ch in range(C)]
            for hh in hs:
                hh.wait()

            @pl.when(kk > 0)
            def _():
                drain_table_write()

            @plsc.parallel_loop(0, P_BUILD // L, unroll=4)
            def tr(g):
                p_idx = g * L + iota
                for ch in range(C):
                    vals = tin_v[ch, pl.ds(g * L, L)]
                    plsc.store_scatter(rows_v, [p_idx, ch_splat[ch]], vals)

            pltpu.async_copy(
                rows_v, table_hbm.at[cid, pl.ds(off, P_BUILD)], sem)
            return carry

        nk = jnp.where(sid < extra, full + 1, full)
        lax.fori_loop(0, nk, build, 0)
        drain_table_write()

        if rem:
            @pl.when(sid == extra)
            def _():
                roff = nfull * P_BUILD
                hs = [pltpu.async_copy(
                          tex_refs[ch].at[pl.ds(roff, rem)],
                          tin_v.at[ch, pl.ds(0, rem)], sem)
                      for ch in range(C)]
                for hh in hs:
                    hh.wait()

                @plsc.parallel_loop(0, rem // L, unroll=4)
                def trr(g):
                    p_idx = g * L + iota
                    for ch in range(C):
                        vals = tin_v[ch, pl.ds(g * L, L)]
                        plsc.store_scatter(rows_v, [p_idx, ch_splat[ch]],
                                           vals)
                pltpu.sync_copy(rows_v.at[pl.ds(0, rem)],
                                table_hbm.at[cid, pl.ds(roff, rem)])

        plsc.subcore_barrier()

        wid = sid * NC + cid
        base = wid * per_w
        bb = wid // (hw // per_w)
        col0 = (wid % (hw // per_w)) * per_w
        h0w = col0 // w

        def drain_out_writes():
            for ch in range(C):
                pltpu.make_async_copy(
                    cm_v2.at[ch],
                    out_hbm.at[bb, ch, pl.ds(h0w, rows_per_chunk), :],
                    sem).wait()

        def gather(j, carry):
            off = base + j * S_CHUNK
            pltpu.sync_copy(ids_hbm.at[pl.ds(off, S_CHUNK)], idx_v)
            pltpu.async_copy(table_hbm.at[cid].at[idx_v], rows_v, sem).wait()

            @pl.when(j > 0)
            def _():
                drain_out_writes()

            gprow = 512 // L
            @plsc.parallel_loop(0, S_CHUNK // L, unroll=4)
            def tr(g):
                s_idx = g * L + iota
                for ch in range(C):
                    vals = plsc.load_gather(rows_v, [s_idx, ch_splat[ch]])
                    cm_v2[ch, g // gprow, pl.ds((g % gprow) * L, L)] = vals

            hrow = h0w + j * rows_per_chunk
            for ch in range(C):
                pltpu.async_copy(
                    cm_v2.at[ch],
                    out_hbm.at[bb, ch, pl.ds(hrow, rows_per_chunk), :],
                    sem)
            return carry

        lax.fori_loop(0, n_g, gather, 0)
        drain_out_writes()

    return k(*tex_cs, ids1d)


def kernel(inputs, texture):
    ids = inputs
    b, h, w = ids.shape
    size = texture.shape[2]
    n = b * h * w
    tex_cs = [texture[0, c] for c in range(C)]
    return _sc_all(tex_cs, ids.reshape(n), n, size, b, h, w)


# speedup vs baseline: 16.8938x; 1.0110x over previous
import functools

import jax
import jax.numpy as jnp
from jax import lax
from jax.experimental import pallas as pl
from jax.experimental.pallas import tpu as pltpu
from jax.experimental.pallas import tpu_sc as plsc

C = 8
NC, NS = 2, 16
NW = NC * NS
L = 16
P_BUILD = 4096
S_CHUNK = 4096


def _sc_all(tex_cs, ids1d, n, size, b, h, w):
    hw = h * w
    per_w = n // NW
    n_g = per_w // S_CHUNK
    nfull = size // P_BUILD
    rem = size - nfull * P_BUILD
    full, extra = nfull // NS, nfull % NS
    rows_per_chunk = S_CHUNK // w
    mesh = plsc.VectorSubcoreMesh(core_axis_name="c", subcore_axis_name="s")

    @functools.partial(
        pl.kernel,
        mesh=mesh,
        out_type=jax.ShapeDtypeStruct((b, C, h, w), jnp.float32),
        scratch_types=[
            pltpu.HBM((NC, size, C), jnp.float32),
            pltpu.VMEM((C, P_BUILD), jnp.float32),
            pltpu.VMEM((S_CHUNK,), jnp.int32),
            pltpu.VMEM((S_CHUNK, C), jnp.float32),
            pltpu.VMEM((C, S_CHUNK // 512, 512), jnp.float32),
            pltpu.SemaphoreType.DMA,
        ],
        compiler_params=pltpu.CompilerParams(use_tc_tiling_on_sc=False,
                                             needs_layout_passes=False),
    )
    def k(t0, t1, t2, t3, t4, t5, t6, t7, ids_hbm, out_hbm, table_hbm,
          tin_v, idx_v, rows_v, cm_v2, sem):
        tex_refs = [t0, t1, t2, t3, t4, t5, t6, t7]
        cid = lax.axis_index("c")
        sid = lax.axis_index("s")
        iota = lax.iota(jnp.int32, L)
        ch_splat = [jnp.full((L,), ch, jnp.int32) for ch in range(C)]

        def drain_table_write():
            pltpu.make_async_copy(
                rows_v, table_hbm.at[cid, pl.ds(0, P_BUILD)], sem).wait()

        def build(kk, carry):
            off = (sid + NS * kk) * P_BUILD
            hs = [pltpu.async_copy(
                      tex_refs[ch].at[pl.ds(off, P_BUILD)],
                      tin_v.at[ch], sem) for ch in range(C)]
            for hh in hs:
                hh.wait()

            @pl.when(kk > 0)
            def _():
                drain_table_write()

            @plsc.parallel_loop(0, P_BUILD // L, unroll=4)
            def tr(g):
                p_idx = g * L + iota
                for ch in range(C):
                    vals = tin_v[ch, pl.ds(g * L, L)]
                    plsc.store_scatter(rows_v, [p_idx, ch_splat[ch]], vals)

            pltpu.async_copy(
                rows_v, table_hbm.at[cid, pl.ds(off, P_BUILD)], sem)
            return carry

        nk = jnp.where(sid < extra, full + 1, full)
        lax.fori_loop(0, nk, build, 0)
        drain_table_write()

        if rem:
            @pl.when(sid == extra)
            def _():
                roff = nfull * P_BUILD
                hs = [pltpu.async_copy(
                          tex_refs[ch].at[pl.ds(roff, rem)],
                          tin_v.at[ch, pl.ds(0, rem)], sem)
                      for ch in range(C)]
                for hh in hs:
                    hh.wait()

                @plsc.parallel_loop(0, rem // L, unroll=4)
                def trr(g):
                    p_idx = g * L + iota
                    for ch in range(C):
                        vals = tin_v[ch, pl.ds(g * L, L)]
                        plsc.store_scatter(rows_v, [p_idx, ch_splat[ch]],
                                           vals)
                pltpu.sync_copy(rows_v.at[pl.ds(0, rem)],
                                table_hbm.at[cid, pl.ds(roff, rem)])

        plsc.subcore_barrier()

        wid = sid * NC + cid
        base = wid * per_w
        bb = wid // (hw // per_w)
        col0 = (wid % (hw // per_w)) * per_w
        h0w = col0 // w

        def drain_out_writes():
            for ch in range(C):
                pltpu.make_async_copy(
                    cm_v2.at[ch],
                    out_hbm.at[bb, ch, pl.ds(h0w, rows_per_chunk), :],
                    sem).wait()

        half = S_CHUNK // 2

        def gather(j, carry):
            off = base + j * S_CHUNK
            pltpu.sync_copy(ids_hbm.at[pl.ds(off, S_CHUNK)], idx_v)
            ha = pltpu.async_copy(
                table_hbm.at[cid].at[idx_v.at[pl.ds(0, half)]],
                rows_v.at[pl.ds(0, half)], sem)
            hb = pltpu.async_copy(
                table_hbm.at[cid].at[idx_v.at[pl.ds(half, half)]],
                rows_v.at[pl.ds(half, half)], sem)
            ha.wait()

            @pl.when(j > 0)
            def _():
                drain_out_writes()

            gprow = 512 // L

            @plsc.parallel_loop(0, half // L, unroll=4)
            def tr(g):
                s_idx = g * L + iota
                for ch in range(C):
                    vals = plsc.load_gather(rows_v, [s_idx, ch_splat[ch]])
                    cm_v2[ch, g // gprow, pl.ds((g % gprow) * L, L)] = vals

            hb.wait()

            @plsc.parallel_loop(half // L, S_CHUNK // L, unroll=4)
            def tr2(g):
                s_idx = g * L + iota
                for ch in range(C):
                    vals = plsc.load_gather(rows_v, [s_idx, ch_splat[ch]])
                    cm_v2[ch, g // gprow, pl.ds((g % gprow) * L, L)] = vals

            hrow = h0w + j * rows_per_chunk
            for ch in range(C):
                pltpu.async_copy(
                    cm_v2.at[ch],
                    out_hbm.at[bb, ch, pl.ds(hrow, rows_per_chunk), :],
                    sem)
            return carry

        lax.fori_loop(0, n_g, gather, 0)
        drain_out_writes()

    return k(*tex_cs, ids1d)


def kernel(inputs, texture):
    ids = inputs
    b, h, w = ids.shape
    size = texture.shape[2]
    n = b * h * w
    tex_cs = [texture[0, c] for c in range(C)]
    return _sc_all(tex_cs, ids.reshape(n), n, size, b, h, w)
